# baseline (speedup 1.0000x reference)
import jax
import jax.numpy as jnp
from jax import lax
from jax.experimental import pallas as pl
from jax.experimental.pallas import tpu as pltpu
from jax.experimental.pallas import tpu_sc as plsc
import functools

BATCH = 16384
DIM = 64
NC = 2
NS = 16
NW = NC * NS
BPW = BATCH // NW
ICH = 128
NCH = BPW // ICH


def _sc_body(uidx_hbm, iidx_hbm, uemb_hbm, iemb_hbm, ubias_hbm, ibias_hbm,
             out_hbm, uidx_v, iidx_v, urows_v, irows_v, ub_v, ib_v, out_v,
             sem):
    wid = lax.axis_index("s") * NC + lax.axis_index("c")
    base = wid * BPW

    pltpu.sync_copy(uidx_hbm.at[wid], uidx_v)
    pltpu.sync_copy(iidx_hbm.at[wid], iidx_v)

    copies = []
    for j in range(NCH):
        dst = pl.ds(j * ICH, ICH)
        copies.append(pltpu.async_copy(uemb_hbm.at[uidx_v.at[j]],
                                       urows_v.at[dst], sem))
        copies.append(pltpu.async_copy(iemb_hbm.at[iidx_v.at[j]],
                                       irows_v.at[dst], sem))
        copies.append(pltpu.async_copy(ubias_hbm.at[uidx_v.at[j]],
                                       ub_v.at[dst], sem))
        copies.append(pltpu.async_copy(ibias_hbm.at[iidx_v.at[j]],
                                       ib_v.at[dst], sem))
    for c in copies:
        c.wait()

    lanes = lax.broadcasted_iota(jnp.int32, (16,), 0)

    def group_body(g, _):
        row_idx = g * 16 + lanes
        accs = [jnp.zeros((16,), jnp.float32) for _ in range(4)]
        for d in range(DIM):
            col = jnp.full((16,), d, jnp.int32)
            cu = plsc.load_gather(urows_v, [row_idx, col])
            ci = plsc.load_gather(irows_v, [row_idx, col])
            accs[d % 4] = accs[d % 4] + cu * ci
        ds = pl.ds(g * 16, 16)
        out_v[ds] = ((accs[0] + accs[1]) + (accs[2] + accs[3])
                     + ub_v[ds] + ib_v[ds])
        return _

    lax.fori_loop(0, BPW // 16, group_body, 0)

    pltpu.sync_copy(out_v, out_hbm.at[pl.ds(base, BPW)])


@jax.jit
def _mf_sc(uidx, iidx, uemb, iemb, ubias, ibias):
    mesh = plsc.VectorSubcoreMesh(core_axis_name="c", subcore_axis_name="s")
    kfn = functools.partial(
        pl.kernel,
        mesh=mesh,
        out_type=jax.ShapeDtypeStruct((BATCH,), jnp.float32),
        scratch_types=[
            pltpu.VMEM((NCH, ICH), jnp.int32),
            pltpu.VMEM((NCH, ICH), jnp.int32),
            pltpu.VMEM((BPW, DIM), jnp.float32),
            pltpu.VMEM((BPW, DIM), jnp.float32),
            pltpu.VMEM((BPW,), jnp.float32),
            pltpu.VMEM((BPW,), jnp.float32),
            pltpu.VMEM((BPW,), jnp.float32),
            pltpu.SemaphoreType.DMA,
        ],
        compiler_params=pltpu.CompilerParams(needs_layout_passes=False,
                                             use_tc_tiling_on_sc=False),
    )(_sc_body)
    return kfn(uidx, iidx, uemb, iemb, ubias, ibias)


def kernel(user_idx, item_idx, user_emb, item_emb, user_bias, item_bias):
    uidx = user_idx.astype(jnp.int32).reshape(NW, NCH, ICH)
    iidx = item_idx.astype(jnp.int32).reshape(NW, NCH, ICH)
    ubias = user_bias.reshape(-1)
    ibias = item_bias.reshape(-1)
    return _mf_sc(uidx, iidx, user_emb, item_emb, ubias, ibias)
